# Initial kernel scaffold; baseline (speedup 1.0000x reference)
import jax
import jax.numpy as jnp
from jax.experimental import pallas as pl


def kernel(x, edge_index, W1, b1, W2, b2):
    raise NotImplementedError("write your pallas kernel here")


# speedup vs baseline: 19.3824x; 19.3824x over previous
import functools

import jax
import jax.numpy as jnp
from jax import lax
from jax.experimental import pallas as pl
from jax.experimental.pallas import tpu as pltpu
from jax.experimental.pallas import tpu_sc as plsc

N = 10000
E = 320000
D = 128
NC = 2
NS = 16
NW = NC * NS
R = 10240
TPR = R // NS
CH = 128
NCHUNK = 79
EPT = CH * NCHUNK
EPAD = NW * EPT

_mesh = plsc.VectorSubcoreMesh(core_axis_name="c", subcore_axis_name="s")


@functools.partial(
    pl.kernel,
    out_type=jax.ShapeDtypeStruct((NC, R, D), jnp.float32),
    mesh=_mesh,
    scratch_types=[
        pltpu.VMEM((NCHUNK, CH), jnp.int32),
        pltpu.VMEM((CH, D), jnp.float32),
        pltpu.VMEM_SHARED((R, D), jnp.float32),
    ],
)
def _deg_kernel(dst_hbm, ones_hbm, zeros_hbm, out_hbm, dst_v, ones_v, acc):
    cid = lax.axis_index("c")
    tid = lax.axis_index("s")
    wid = cid * NS + tid
    pltpu.sync_copy(zeros_hbm, acc.at[pl.ds(tid * TPR, TPR)])
    pltpu.sync_copy(ones_hbm, ones_v)
    pltpu.sync_copy(dst_hbm.at[wid], dst_v)
    plsc.subcore_barrier()

    def body(j, carry):
        pltpu.sync_copy(ones_v, acc.at[dst_v.at[j]], add=True)
        return carry

    lax.fori_loop(0, NCHUNK, body, 0)
    plsc.subcore_barrier()
    pltpu.sync_copy(acc.at[pl.ds(tid * TPR, TPR)],
                    out_hbm.at[cid, pl.ds(tid * TPR, TPR)])


@functools.partial(
    pl.kernel,
    out_type=jax.ShapeDtypeStruct((NC, R, D), jnp.float32),
    mesh=_mesh,
    scratch_types=[
        pltpu.VMEM((NCHUNK, CH), jnp.int32),
        pltpu.VMEM((NCHUNK, CH), jnp.int32),
        pltpu.VMEM((CH, D), jnp.float32),
        pltpu.VMEM_SHARED((R, D), jnp.float32),
        pltpu.SemaphoreType.DMA,
    ],
)
def _scat_kernel(g_hbm, src_hbm, dst_hbm, zeros_hbm, out_hbm,
                 src_v, dst_v, rows_v, acc, sem):
    cid = lax.axis_index("c")
    tid = lax.axis_index("s")
    wid = cid * NS + tid
    pltpu.sync_copy(zeros_hbm, acc.at[pl.ds(tid * TPR, TPR)])
    pltpu.sync_copy(src_hbm.at[wid], src_v)
    pltpu.sync_copy(dst_hbm.at[wid], dst_v)
    plsc.subcore_barrier()

    def body(j, carry):
        pltpu.async_copy(g_hbm.at[src_v.at[j]], rows_v, sem).wait()
        pltpu.sync_copy(rows_v, acc.at[dst_v.at[j]], add=True)
        return carry

    lax.fori_loop(0, NCHUNK, body, 0)
    plsc.subcore_barrier()
    pltpu.sync_copy(acc.at[pl.ds(tid * TPR, TPR)],
                    out_hbm.at[cid, pl.ds(tid * TPR, TPR)])


_B = 512
_G = pl.cdiv(N, _B)


def _dis_of(degp_ref):
    d = degp_ref[0, :, 0:1] + degp_ref[1, :, 0:1] + 1.0
    return lax.rsqrt(d)


def _mm1_body(degp_ref, x_ref, w_ref, o_ref):
    dis = _dis_of(degp_ref)
    o_ref[...] = jnp.dot(x_ref[...], w_ref[...],
                         preferred_element_type=jnp.float32) * dis


def _mm2_body(degp_ref, s_ref, g_ref, b_ref, w_ref, o_ref):
    dis = _dis_of(degp_ref)
    t = s_ref[0] + s_ref[1] + g_ref[...]
    h = jnp.maximum(t * dis + b_ref[...], 0.0)
    o_ref[...] = jnp.dot(h, w_ref[...],
                         preferred_element_type=jnp.float32) * dis


def _mm3_body(degp_ref, t_ref, g_ref, b_ref, o_ref):
    dis = _dis_of(degp_ref)
    o_ref[...] = (t_ref[0] + t_ref[1] + g_ref[...]) * dis + b_ref[...]


_degp_spec = pl.BlockSpec((2, _B, D), lambda i: (0, i, 0))
_acc_spec = pl.BlockSpec((2, _B, D), lambda i: (0, i, 0))
_row_spec = pl.BlockSpec((_B, D), lambda i: (i, 0))
_w_spec = pl.BlockSpec((D, D), lambda i: (0, 0))
_b_spec = pl.BlockSpec((1, D), lambda i: (0, 0))
_out_nd = jax.ShapeDtypeStruct((N, D), jnp.float32)

_mm1 = pl.pallas_call(
    _mm1_body, grid=(_G,),
    in_specs=[_degp_spec, _row_spec, _w_spec],
    out_specs=_row_spec, out_shape=_out_nd)

_mm2 = pl.pallas_call(
    _mm2_body, grid=(_G,),
    in_specs=[_degp_spec, _acc_spec, _row_spec, _b_spec, _w_spec],
    out_specs=_row_spec, out_shape=_out_nd)

_mm3 = pl.pallas_call(
    _mm3_body, grid=(_G,),
    in_specs=[_degp_spec, _acc_spec, _row_spec, _b_spec],
    out_specs=_row_spec, out_shape=_out_nd)


def kernel(x, edge_index, W1, b1, W2, b2):
    src = edge_index[0]
    dst = edge_index[1]
    npad = EPAD - E
    ar = jnp.arange(npad, dtype=jnp.int32)
    pad_src = ar % N
    pad_dst = N + ar % (R - N)
    srcp = jnp.concatenate([src, pad_src]).reshape(NW, NCHUNK, CH)
    dstp = jnp.concatenate([dst, pad_dst]).reshape(NW, NCHUNK, CH)

    onesd = jnp.ones((CH, D), jnp.float32)
    zerosd = jnp.zeros((TPR, D), jnp.float32)
    b1r = b1.reshape(1, D)
    b2r = b2.reshape(1, D)

    degp = _deg_kernel(dstp, onesd, zerosd)
    g1 = _mm1(degp, x, W1)
    s1 = _scat_kernel(g1, srcp, dstp, zerosd)
    g2 = _mm2(degp, s1, g1, b1r, W2)
    s2 = _scat_kernel(g2, srcp, dstp, zerosd)
    return _mm3(degp, s2, g2, b2r)


# speedup vs baseline: 25.9446x; 1.3386x over previous
import functools

import jax
import jax.numpy as jnp
from jax import lax
from jax.experimental import pallas as pl
from jax.experimental.pallas import tpu as pltpu
from jax.experimental.pallas import tpu_sc as plsc

N = 10000
E = 320000
D = 128
NC = 2
NS = 16
NW = NC * NS
R = 10240
TPR = R // NS
CH = 128
NCHUNK = 80
HALF = NCHUNK // 2
EPT = CH * NCHUNK
EPAD = NW * EPT

_mesh = plsc.VectorSubcoreMesh(core_axis_name="c", subcore_axis_name="s")


@functools.partial(
    pl.kernel,
    out_type=jax.ShapeDtypeStruct((NC, R, D), jnp.float32),
    mesh=_mesh,
    scratch_types=[
        pltpu.VMEM((NCHUNK, CH), jnp.int32),
        pltpu.VMEM((CH, D), jnp.float32),
        pltpu.VMEM_SHARED((R, D), jnp.float32),
    ],
)
def _deg_kernel(dst_hbm, ones_hbm, zeros_hbm, out_hbm, dst_v, ones_v, acc):
    cid = lax.axis_index("c")
    tid = lax.axis_index("s")
    wid = cid * NS + tid
    pltpu.sync_copy(zeros_hbm, acc.at[pl.ds(tid * TPR, TPR)])
    pltpu.sync_copy(ones_hbm, ones_v)
    pltpu.sync_copy(dst_hbm.at[wid], dst_v)
    plsc.subcore_barrier()

    def body(j, carry):
        pltpu.sync_copy(ones_v, acc.at[dst_v.at[j]], add=True)
        return carry

    lax.fori_loop(0, NCHUNK, body, 0)
    plsc.subcore_barrier()
    pltpu.sync_copy(acc.at[pl.ds(tid * TPR, TPR)],
                    out_hbm.at[cid, pl.ds(tid * TPR, TPR)])


@functools.partial(
    pl.kernel,
    out_type=jax.ShapeDtypeStruct((NC, R, D), jnp.float32),
    mesh=_mesh,
    scratch_types=[
        pltpu.VMEM((NCHUNK, CH), jnp.int32),
        pltpu.VMEM((2, CH, D), jnp.float32),
        pltpu.VMEM((2, CH), jnp.int32),
        pltpu.VMEM_SHARED((R, D), jnp.float32),
        pltpu.SemaphoreType.DMA,
        pltpu.SemaphoreType.DMA,
        pltpu.SemaphoreType.DMA,
        pltpu.SemaphoreType.DMA,
    ],
)
def _scat_kernel(g_hbm, src_hbm, dst_hbm, zeros_hbm, out_hbm,
                 src_v, rows_v, didx_v, acc, sem_a, sem_b, sem_c, sem_d):
    cid = lax.axis_index("c")
    tid = lax.axis_index("s")
    wid = cid * NS + tid
    pltpu.sync_copy(zeros_hbm, acc.at[pl.ds(tid * TPR, TPR)])
    pltpu.sync_copy(src_hbm.at[wid], src_v)
    plsc.subcore_barrier()

    pltpu.async_copy(g_hbm.at[src_v.at[0]], rows_v.at[0], sem_a)
    pltpu.async_copy(dst_hbm.at[wid, 0], didx_v.at[0], sem_c)

    def body(i, carry):
        j0 = 2 * i
        j1 = 2 * i + 1
        pltpu.async_copy(g_hbm.at[src_v.at[j1]], rows_v.at[1], sem_b)
        pltpu.async_copy(dst_hbm.at[wid, j1], didx_v.at[1], sem_d)
        pltpu.make_async_copy(g_hbm.at[src_v.at[j0]], rows_v.at[0],
                              sem_a).wait()
        pltpu.make_async_copy(dst_hbm.at[wid, j0], didx_v.at[0],
                              sem_c).wait()
        pltpu.sync_copy(rows_v.at[0], acc.at[didx_v.at[0]], add=True)

        @pl.when(i + 1 < HALF)
        def _():
            pltpu.async_copy(g_hbm.at[src_v.at[j0 + 2]], rows_v.at[0], sem_a)
            pltpu.async_copy(dst_hbm.at[wid, j0 + 2], didx_v.at[0], sem_c)

        pltpu.make_async_copy(g_hbm.at[src_v.at[j1]], rows_v.at[1],
                              sem_b).wait()
        pltpu.make_async_copy(dst_hbm.at[wid, j1], didx_v.at[1],
                              sem_d).wait()
        pltpu.sync_copy(rows_v.at[1], acc.at[didx_v.at[1]], add=True)
        return carry

    lax.fori_loop(0, HALF, body, 0)
    plsc.subcore_barrier()
    pltpu.sync_copy(acc.at[pl.ds(tid * TPR, TPR)],
                    out_hbm.at[cid, pl.ds(tid * TPR, TPR)])


_B = 512
_G = pl.cdiv(N, _B)


def _dis_of(degp_ref):
    d = degp_ref[0, :, 0:1] + degp_ref[1, :, 0:1] + 1.0
    return lax.rsqrt(d)


def _mm1_body(degp_ref, x_ref, w_ref, o_ref):
    dis = _dis_of(degp_ref)
    o_ref[...] = jnp.dot(x_ref[...], w_ref[...],
                         preferred_element_type=jnp.float32) * dis


def _mm2_body(degp_ref, s_ref, g_ref, b_ref, w_ref, o_ref):
    dis = _dis_of(degp_ref)
    t = s_ref[0] + s_ref[1] + g_ref[...]
    h = jnp.maximum(t * dis + b_ref[...], 0.0)
    o_ref[...] = jnp.dot(h, w_ref[...],
                         preferred_element_type=jnp.float32) * dis


def _mm3_body(degp_ref, t_ref, g_ref, b_ref, o_ref):
    dis = _dis_of(degp_ref)
    o_ref[...] = (t_ref[0] + t_ref[1] + g_ref[...]) * dis + b_ref[...]


_degp_spec = pl.BlockSpec((2, _B, D), lambda i: (0, i, 0))
_acc_spec = pl.BlockSpec((2, _B, D), lambda i: (0, i, 0))
_row_spec = pl.BlockSpec((_B, D), lambda i: (i, 0))
_w_spec = pl.BlockSpec((D, D), lambda i: (0, 0))
_b_spec = pl.BlockSpec((1, D), lambda i: (0, 0))
_out_nd = jax.ShapeDtypeStruct((N, D), jnp.float32)

_mm1 = pl.pallas_call(
    _mm1_body, grid=(_G,),
    in_specs=[_degp_spec, _row_spec, _w_spec],
    out_specs=_row_spec, out_shape=_out_nd)

_mm2 = pl.pallas_call(
    _mm2_body, grid=(_G,),
    in_specs=[_degp_spec, _acc_spec, _row_spec, _b_spec, _w_spec],
    out_specs=_row_spec, out_shape=_out_nd)

_mm3 = pl.pallas_call(
    _mm3_body, grid=(_G,),
    in_specs=[_degp_spec, _acc_spec, _row_spec, _b_spec],
    out_specs=_row_spec, out_shape=_out_nd)


def kernel(x, edge_index, W1, b1, W2, b2):
    src = edge_index[0]
    dst = edge_index[1]
    npad = EPAD - E
    ar = jnp.arange(npad, dtype=jnp.int32)
    pad_src = ar % N
    pad_dst = N + ar % (R - N)
    srcp = jnp.concatenate([src, pad_src]).reshape(NW, NCHUNK, CH)
    dstp = jnp.concatenate([dst, pad_dst]).reshape(NW, NCHUNK, CH)

    onesd = jnp.ones((CH, D), jnp.float32)
    zerosd = jnp.zeros((TPR, D), jnp.float32)
    b1r = b1.reshape(1, D)
    b2r = b2.reshape(1, D)

    degp = _deg_kernel(dstp, onesd, zerosd)
    g1 = _mm1(degp, x, W1)
    s1 = _scat_kernel(g1, srcp, dstp, zerosd)
    g2 = _mm2(degp, s1, g1, b1r, W2)
    s2 = _scat_kernel(g2, srcp, dstp, zerosd)
    return _mm3(degp, s2, g2, b2r)


# speedup vs baseline: 28.5189x; 1.0992x over previous
import functools

import jax
import jax.numpy as jnp
from jax import lax
from jax.experimental import pallas as pl
from jax.experimental.pallas import tpu as pltpu
from jax.experimental.pallas import tpu_sc as plsc

N = 10000
E = 320000
D = 128
NC = 2
NS = 16
NW = NC * NS
R = 10240
TPR = R // NS
CH = 128
NCHUNK = 80
EPT = CH * NCHUNK
TAILC = (E - (NW - 1) * EPT) // CH

_mesh = plsc.VectorSubcoreMesh(core_axis_name="c", subcore_axis_name="s")


def _fill(ref, val):
    v = jnp.full((16,), val, jnp.float32)

    def body(r, carry):
        for k in range(D // 16):
            ref[r, pl.ds(k * 16, 16)] = v
        return carry

    lax.fori_loop(0, CH, body, 0)


def _chunk_lim(wid):
    return jnp.where(wid == NW - 1, TAILC, NCHUNK)


@functools.partial(
    pl.kernel,
    out_type=jax.ShapeDtypeStruct((NC, R, D), jnp.float32),
    mesh=_mesh,
    scratch_types=[
        pltpu.VMEM((CH, D), jnp.float32),
        pltpu.VMEM((2, CH), jnp.int32),
        pltpu.VMEM_SHARED((R, D), jnp.float32),
        pltpu.SemaphoreType.DMA,
        pltpu.SemaphoreType.DMA,
    ],
)
def _deg_kernel(ei_hbm, out_hbm, ones_v, didx_v, acc, sem_c, sem_d):
    cid = lax.axis_index("c")
    tid = lax.axis_index("s")
    wid = cid * NS + tid
    base = wid * EPT
    hl = _chunk_lim(wid) // 2

    _fill(ones_v, 0.0)
    for k in range(TPR // CH):
        pltpu.sync_copy(ones_v, acc.at[pl.ds(tid * TPR + k * CH, CH)])
    _fill(ones_v, 1.0)
    plsc.subcore_barrier()

    pltpu.async_copy(ei_hbm.at[1, pl.ds(base, CH)], didx_v.at[0], sem_c)

    def body(i, carry):
        j0 = 2 * i
        j1 = 2 * i + 1
        pltpu.async_copy(ei_hbm.at[1, pl.ds(base + j1 * CH, CH)],
                         didx_v.at[1], sem_d)
        pltpu.make_async_copy(ei_hbm.at[1, pl.ds(base + j0 * CH, CH)],
                              didx_v.at[0], sem_c).wait()
        pltpu.sync_copy(ones_v, acc.at[didx_v.at[0]], add=True)

        @pl.when(i + 1 < hl)
        def _():
            pltpu.async_copy(ei_hbm.at[1, pl.ds(base + (j0 + 2) * CH, CH)],
                             didx_v.at[0], sem_c)

        pltpu.make_async_copy(ei_hbm.at[1, pl.ds(base + j1 * CH, CH)],
                              didx_v.at[1], sem_d).wait()
        pltpu.sync_copy(ones_v, acc.at[didx_v.at[1]], add=True)
        return carry

    lax.fori_loop(0, hl, body, 0)
    plsc.subcore_barrier()
    pltpu.sync_copy(acc.at[pl.ds(tid * TPR, TPR)],
                    out_hbm.at[cid, pl.ds(tid * TPR, TPR)])


@functools.partial(
    pl.kernel,
    out_type=jax.ShapeDtypeStruct((NC, R, D), jnp.float32),
    mesh=_mesh,
    scratch_types=[
        pltpu.VMEM((EPT,), jnp.int32),
        pltpu.VMEM((CH, D), jnp.float32),
        pltpu.VMEM((CH, D), jnp.float32),
        pltpu.VMEM((2, CH), jnp.int32),
        pltpu.VMEM_SHARED((R, D), jnp.float32),
        pltpu.SemaphoreType.DMA,
        pltpu.SemaphoreType.DMA,
        pltpu.SemaphoreType.DMA,
        pltpu.SemaphoreType.DMA,
    ],
)
def _scat_kernel(g_hbm, ei_hbm, out_hbm,
                 src_v, rows0_v, rows1_v, didx_v, acc,
                 sem_a, sem_b, sem_c, sem_d):
    cid = lax.axis_index("c")
    tid = lax.axis_index("s")
    wid = cid * NS + tid
    base = wid * EPT
    hl = _chunk_lim(wid) // 2

    _fill(rows0_v, 0.0)
    for k in range(TPR // CH):
        pltpu.sync_copy(rows0_v, acc.at[pl.ds(tid * TPR + k * CH, CH)])

    @pl.when(wid < NW - 1)
    def _():
        pltpu.sync_copy(ei_hbm.at[0, pl.ds(base, EPT)], src_v)

    @pl.when(wid == NW - 1)
    def _():
        pltpu.sync_copy(ei_hbm.at[0, pl.ds((NW - 1) * EPT, TAILC * CH)],
                        src_v.at[pl.ds(0, TAILC * CH)])

    plsc.subcore_barrier()

    pltpu.async_copy(g_hbm.at[src_v.at[pl.ds(0, CH)]], rows0_v, sem_a)
    pltpu.async_copy(ei_hbm.at[1, pl.ds(base, CH)], didx_v.at[0], sem_c)

    def body(i, carry):
        j0 = 2 * i
        j1 = 2 * i + 1
        pltpu.async_copy(g_hbm.at[src_v.at[pl.ds(j1 * CH, CH)]], rows1_v,
                         sem_b)
        pltpu.async_copy(ei_hbm.at[1, pl.ds(base + j1 * CH, CH)],
                         didx_v.at[1], sem_d)
        pltpu.make_async_copy(g_hbm.at[src_v.at[pl.ds(j0 * CH, CH)]],
                              rows0_v, sem_a).wait()
        pltpu.make_async_copy(ei_hbm.at[1, pl.ds(base + j0 * CH, CH)],
                              didx_v.at[0], sem_c).wait()
        pltpu.sync_copy(rows0_v, acc.at[didx_v.at[0]], add=True)

        @pl.when(i + 1 < hl)
        def _():
            pltpu.async_copy(g_hbm.at[src_v.at[pl.ds((j0 + 2) * CH, CH)]],
                             rows0_v, sem_a)
            pltpu.async_copy(ei_hbm.at[1, pl.ds(base + (j0 + 2) * CH, CH)],
                             didx_v.at[0], sem_c)

        pltpu.make_async_copy(g_hbm.at[src_v.at[pl.ds(j1 * CH, CH)]],
                              rows1_v, sem_b).wait()
        pltpu.make_async_copy(ei_hbm.at[1, pl.ds(base + j1 * CH, CH)],
                              didx_v.at[1], sem_d).wait()
        pltpu.sync_copy(rows1_v, acc.at[didx_v.at[1]], add=True)
        return carry

    lax.fori_loop(0, hl, body, 0)
    plsc.subcore_barrier()
    pltpu.sync_copy(acc.at[pl.ds(tid * TPR, TPR)],
                    out_hbm.at[cid, pl.ds(tid * TPR, TPR)])


_B = 512
_G = pl.cdiv(N, _B)


def _dis_of(degp_ref):
    d = degp_ref[0, :, 0:1] + degp_ref[1, :, 0:1] + 1.0
    return lax.rsqrt(d)


def _mm1_body(degp_ref, x_ref, w_ref, o_ref):
    dis = _dis_of(degp_ref)
    o_ref[...] = jnp.dot(x_ref[...], w_ref[...],
                         preferred_element_type=jnp.float32) * dis


def _mm2_body(degp_ref, s_ref, g_ref, b_ref, w_ref, o_ref):
    dis = _dis_of(degp_ref)
    t = s_ref[0] + s_ref[1] + g_ref[...]
    h = jnp.maximum(t * dis + b_ref[...], 0.0)
    o_ref[...] = jnp.dot(h, w_ref[...],
                         preferred_element_type=jnp.float32) * dis


def _mm3_body(degp_ref, t_ref, g_ref, b_ref, o_ref):
    dis = _dis_of(degp_ref)
    o_ref[...] = (t_ref[0] + t_ref[1] + g_ref[...]) * dis + b_ref[...]


_degp_spec = pl.BlockSpec((2, _B, D), lambda i: (0, i, 0))
_acc_spec = pl.BlockSpec((2, _B, D), lambda i: (0, i, 0))
_row_spec = pl.BlockSpec((_B, D), lambda i: (i, 0))
_w_spec = pl.BlockSpec((D, D), lambda i: (0, 0))
_b_spec = pl.BlockSpec((1, D), lambda i: (0, 0))
_out_nd = jax.ShapeDtypeStruct((N, D), jnp.float32)

_mm1 = pl.pallas_call(
    _mm1_body, grid=(_G,),
    in_specs=[_degp_spec, _row_spec, _w_spec],
    out_specs=_row_spec, out_shape=_out_nd)

_mm2 = pl.pallas_call(
    _mm2_body, grid=(_G,),
    in_specs=[_degp_spec, _acc_spec, _row_spec, _b_spec, _w_spec],
    out_specs=_row_spec, out_shape=_out_nd)

_mm3 = pl.pallas_call(
    _mm3_body, grid=(_G,),
    in_specs=[_degp_spec, _acc_spec, _row_spec, _b_spec],
    out_specs=_row_spec, out_shape=_out_nd)


def kernel(x, edge_index, W1, b1, W2, b2):
    b1r = b1.reshape(1, D)
    b2r = b2.reshape(1, D)

    degp = _deg_kernel(edge_index)
    g1 = _mm1(degp, x, W1)
    s1 = _scat_kernel(g1, edge_index)
    g2 = _mm2(degp, s1, g1, b1r, W2)
    s2 = _scat_kernel(g2, edge_index)
    return _mm3(degp, s2, g2, b2r)


# speedup vs baseline: 31.0251x; 1.0879x over previous
import functools

import jax
import jax.numpy as jnp
from jax import lax
from jax.experimental import pallas as pl
from jax.experimental.pallas import tpu as pltpu
from jax.experimental.pallas import tpu_sc as plsc

N = 10000
E = 320000
D = 128
NC = 2
NS = 16
NW = NC * NS
R = 10240
TPR = R // NS
CH = 128
NCHUNK = 80
EPT = CH * NCHUNK
TAILC = (E - (NW - 1) * EPT) // CH

DW = 32

_mesh = plsc.VectorSubcoreMesh(core_axis_name="c", subcore_axis_name="s")


def _fill(ref, val, width):
    v = jnp.full((16,), val, jnp.float32)

    def body(r, carry):
        for k in range(width // 16):
            ref[r, pl.ds(k * 16, 16)] = v
        return carry

    lax.fori_loop(0, CH, body, 0)


def _chunk_lim(wid):
    return jnp.where(wid == NW - 1, TAILC, NCHUNK)


@functools.partial(
    pl.kernel,
    out_type=jax.ShapeDtypeStruct((NC, R, DW), jnp.float32),
    mesh=_mesh,
    scratch_types=[
        pltpu.VMEM((CH, DW), jnp.float32),
        pltpu.VMEM((2, CH), jnp.int32),
        pltpu.VMEM_SHARED((R, DW), jnp.float32),
        pltpu.SemaphoreType.DMA,
        pltpu.SemaphoreType.DMA,
    ],
)
def _deg_kernel(ei_hbm, out_hbm, ones_v, didx_v, acc, sem_c, sem_d):
    cid = lax.axis_index("c")
    tid = lax.axis_index("s")
    wid = cid * NS + tid
    base = wid * EPT
    hl = _chunk_lim(wid) // 2

    _fill(ones_v, 0.0, DW)
    for k in range(TPR // CH):
        pltpu.sync_copy(ones_v, acc.at[pl.ds(tid * TPR + k * CH, CH)])
    _fill(ones_v, 1.0, DW)
    plsc.subcore_barrier()

    pltpu.async_copy(ei_hbm.at[1, pl.ds(base, CH)], didx_v.at[0], sem_c)

    def body(i, carry):
        j0 = 2 * i
        j1 = 2 * i + 1
        pltpu.async_copy(ei_hbm.at[1, pl.ds(base + j1 * CH, CH)],
                         didx_v.at[1], sem_d)
        pltpu.make_async_copy(ei_hbm.at[1, pl.ds(base + j0 * CH, CH)],
                              didx_v.at[0], sem_c).wait()
        pltpu.sync_copy(ones_v, acc.at[didx_v.at[0]], add=True)

        @pl.when(i + 1 < hl)
        def _():
            pltpu.async_copy(ei_hbm.at[1, pl.ds(base + (j0 + 2) * CH, CH)],
                             didx_v.at[0], sem_c)

        pltpu.make_async_copy(ei_hbm.at[1, pl.ds(base + j1 * CH, CH)],
                              didx_v.at[1], sem_d).wait()
        pltpu.sync_copy(ones_v, acc.at[didx_v.at[1]], add=True)
        return carry

    lax.fori_loop(0, hl, body, 0)
    plsc.subcore_barrier()
    pltpu.sync_copy(acc.at[pl.ds(tid * TPR, TPR)],
                    out_hbm.at[cid, pl.ds(tid * TPR, TPR)])


@functools.partial(
    pl.kernel,
    out_type=jax.ShapeDtypeStruct((NC, R, D), jnp.float32),
    mesh=_mesh,
    scratch_types=[
        pltpu.VMEM((EPT,), jnp.int32),
        pltpu.VMEM((CH, D), jnp.float32),
        pltpu.VMEM((CH, D), jnp.float32),
        pltpu.VMEM((2, CH), jnp.int32),
        pltpu.VMEM_SHARED((R, D), jnp.float32),
        pltpu.SemaphoreType.DMA,
        pltpu.SemaphoreType.DMA,
        pltpu.SemaphoreType.DMA,
        pltpu.SemaphoreType.DMA,
    ],
)
def _scat_kernel(g_hbm, ei_hbm, out_hbm,
                 src_v, rows0_v, rows1_v, didx_v, acc,
                 sem_a, sem_b, sem_c, sem_d):
    cid = lax.axis_index("c")
    tid = lax.axis_index("s")
    wid = cid * NS + tid
    base = wid * EPT
    hl = _chunk_lim(wid) // 2

    _fill(rows0_v, 0.0, D)
    for k in range(TPR // CH):
        pltpu.sync_copy(rows0_v, acc.at[pl.ds(tid * TPR + k * CH, CH)])

    @pl.when(wid < NW - 1)
    def _():
        pltpu.sync_copy(ei_hbm.at[0, pl.ds(base, EPT)], src_v)

    @pl.when(wid == NW - 1)
    def _():
        pltpu.sync_copy(ei_hbm.at[0, pl.ds((NW - 1) * EPT, TAILC * CH)],
                        src_v.at[pl.ds(0, TAILC * CH)])

    plsc.subcore_barrier()

    pltpu.async_copy(g_hbm.at[src_v.at[pl.ds(0, CH)]], rows0_v, sem_a)
    pltpu.async_copy(ei_hbm.at[1, pl.ds(base, CH)], didx_v.at[0], sem_c)

    def body(i, carry):
        j0 = 2 * i
        j1 = 2 * i + 1
        pltpu.async_copy(g_hbm.at[src_v.at[pl.ds(j1 * CH, CH)]], rows1_v,
                         sem_b)
        pltpu.async_copy(ei_hbm.at[1, pl.ds(base + j1 * CH, CH)],
                         didx_v.at[1], sem_d)
        pltpu.make_async_copy(g_hbm.at[src_v.at[pl.ds(j0 * CH, CH)]],
                              rows0_v, sem_a).wait()
        pltpu.make_async_copy(ei_hbm.at[1, pl.ds(base + j0 * CH, CH)],
                              didx_v.at[0], sem_c).wait()
        pltpu.sync_copy(rows0_v, acc.at[didx_v.at[0]], add=True)

        @pl.when(i + 1 < hl)
        def _():
            pltpu.async_copy(g_hbm.at[src_v.at[pl.ds((j0 + 2) * CH, CH)]],
                             rows0_v, sem_a)
            pltpu.async_copy(ei_hbm.at[1, pl.ds(base + (j0 + 2) * CH, CH)],
                             didx_v.at[0], sem_c)

        pltpu.make_async_copy(g_hbm.at[src_v.at[pl.ds(j1 * CH, CH)]],
                              rows1_v, sem_b).wait()
        pltpu.make_async_copy(ei_hbm.at[1, pl.ds(base + j1 * CH, CH)],
                              didx_v.at[1], sem_d).wait()
        pltpu.sync_copy(rows1_v, acc.at[didx_v.at[1]], add=True)
        return carry

    lax.fori_loop(0, hl, body, 0)
    plsc.subcore_barrier()
    pltpu.sync_copy(acc.at[pl.ds(tid * TPR, TPR)],
                    out_hbm.at[cid, pl.ds(tid * TPR, TPR)])


_B = 512
_G = pl.cdiv(N, _B)


def _dis_of(degp_ref):
    d = degp_ref[0, :, 0:1] + degp_ref[1, :, 0:1] + 1.0
    return lax.rsqrt(d)


def _mm1_body(degp_ref, x_ref, w_ref, o_ref):
    dis = _dis_of(degp_ref)
    o_ref[...] = jnp.dot(x_ref[...], w_ref[...],
                         preferred_element_type=jnp.float32) * dis


def _mm2_body(degp_ref, s_ref, g_ref, b_ref, w_ref, o_ref):
    dis = _dis_of(degp_ref)
    t = s_ref[0] + s_ref[1] + g_ref[...]
    h = jnp.maximum(t * dis + b_ref[...], 0.0)
    o_ref[...] = jnp.dot(h, w_ref[...],
                         preferred_element_type=jnp.float32) * dis


def _mm3_body(degp_ref, t_ref, g_ref, b_ref, o_ref):
    dis = _dis_of(degp_ref)
    o_ref[...] = (t_ref[0] + t_ref[1] + g_ref[...]) * dis + b_ref[...]


_degp_spec = pl.BlockSpec((2, _B, DW), lambda i: (0, i, 0))
_acc_spec = pl.BlockSpec((2, _B, D), lambda i: (0, i, 0))
_row_spec = pl.BlockSpec((_B, D), lambda i: (i, 0))
_w_spec = pl.BlockSpec((D, D), lambda i: (0, 0))
_b_spec = pl.BlockSpec((1, D), lambda i: (0, 0))
_out_nd = jax.ShapeDtypeStruct((N, D), jnp.float32)

_mm1 = pl.pallas_call(
    _mm1_body, grid=(_G,),
    in_specs=[_degp_spec, _row_spec, _w_spec],
    out_specs=_row_spec, out_shape=_out_nd)

_mm2 = pl.pallas_call(
    _mm2_body, grid=(_G,),
    in_specs=[_degp_spec, _acc_spec, _row_spec, _b_spec, _w_spec],
    out_specs=_row_spec, out_shape=_out_nd)

_mm3 = pl.pallas_call(
    _mm3_body, grid=(_G,),
    in_specs=[_degp_spec, _acc_spec, _row_spec, _b_spec],
    out_specs=_row_spec, out_shape=_out_nd)


def kernel(x, edge_index, W1, b1, W2, b2):
    b1r = b1.reshape(1, D)
    b2r = b2.reshape(1, D)

    degp = _deg_kernel(edge_index)
    g1 = _mm1(degp, x, W1)
    s1 = _scat_kernel(g1, edge_index)
    g2 = _mm2(degp, s1, g1, b1r, W2)
    s2 = _scat_kernel(g2, edge_index)
    return _mm3(degp, s2, g2, b2r)
